# Initial kernel scaffold; baseline (speedup 1.0000x reference)
import jax
import jax.numpy as jnp
from jax.experimental import pallas as pl


def kernel(y_pred, y_true):
    raise NotImplementedError("write your pallas kernel here")


# speedup vs baseline: 1.3094x; 1.3094x over previous
import functools

import jax
import jax.numpy as jnp
from jax import lax
from jax.experimental import pallas as pl
from jax.experimental.pallas import tpu as pltpu
from jax.experimental.pallas import tpu_sc as plsc

_K = 5
_IGNORE = -100
_ROWS = 128
_COLS = 100000
_LANES = 16
_UNROLL = 5
_NSTEP = _COLS // (_LANES * _UNROLL)
_NC = 2
_NS = 16
_NW = _NC * _NS
_ROWS_PER = _ROWS // _NW


def _body(ypred_hbm, ytrue_hbm, hits_hbm, wsum_hbm, yt_v, row_v, hit_v, w_v):
    wid = lax.axis_index("s") * _NC + lax.axis_index("c")
    pltpu.sync_copy(ytrue_hbm, yt_v)

    zero_f = jnp.zeros((_LANES,), jnp.float32)
    hit_acc = zero_f
    w_acc = zero_f
    lane_iota = lax.iota(jnp.int32, _LANES)

    for r in range(_ROWS_PER):
        row = wid * _ROWS_PER + r
        pltpu.sync_copy(ypred_hbm.at[row], row_v)
        row_vec = jnp.full((_LANES,), row, jnp.int32)
        t_vec = plsc.load_gather(yt_v, [row_vec])
        t_idx = jnp.clip(t_vec, 0, _COLS - 1)
        v = plsc.load_gather(row_v, [t_idx])

        def step(w, carry, _t=t_vec, _v=v):
            acc, col = carry
            base = w * (_LANES * _UNROLL)
            for u in range(_UNROLL):
                x = row_v[pl.ds(base + u * _LANES, _LANES)]
                m = (x > _v) | ((x == _v) & (col < _t))
                acc = acc + plsc.all_reduce_population_count(m)
                col = col + _LANES
            return acc, col

        acc0 = jnp.zeros((_LANES,), jnp.int32)
        count, _ = lax.fori_loop(0, _NSTEP, step, (acc0, lane_iota))

        valid = t_vec != _IGNORE
        wf = jnp.where(valid, 1.0, 0.0).astype(jnp.float32)
        hit = jnp.where((count < _K) & valid, 1.0, 0.0).astype(jnp.float32)
        hit_acc = hit_acc + hit
        w_acc = w_acc + wf

    hit_v[...] = hit_acc
    w_v[...] = w_acc
    pltpu.sync_copy(hit_v, hits_hbm.at[wid])
    pltpu.sync_copy(w_v, wsum_hbm.at[wid])


@jax.jit
def kernel(y_pred, y_true):
    mesh = plsc.VectorSubcoreMesh(core_axis_name="c", subcore_axis_name="s")
    f = functools.partial(
        pl.kernel,
        mesh=mesh,
        compiler_params=pltpu.CompilerParams(needs_layout_passes=False),
        out_type=[
            jax.ShapeDtypeStruct((_NW, _LANES), jnp.float32),
            jax.ShapeDtypeStruct((_NW, _LANES), jnp.float32),
        ],
        scratch_types=[
            pltpu.VMEM((_ROWS,), jnp.int32),
            pltpu.VMEM((_COLS,), jnp.float32),
            pltpu.VMEM((_LANES,), jnp.float32),
            pltpu.VMEM((_LANES,), jnp.float32),
        ],
    )(_body)
    hits, ws = f(y_pred, y_true.astype(jnp.int32))
    return (hits[:, 0].sum() / ws[:, 0].sum()) * 100.0
